# baseline (speedup 1.0000x reference)
import jax
import jax.numpy as jnp
from jax import lax
from jax.experimental import pallas as pl
from jax.experimental.pallas import tpu as pltpu
from jax.experimental.pallas import tpu_sc as plsc

VOCAB = 100000
DIM = 64
BATCH = 4096
SEQ = 200
TAGD = 8
OUTD = DIM + TAGD + 1

NC = 2
NS = 16
NW = NC * NS
CB = 128
LG = 8
NT = SEQ // LG
NG = CB // 16
CTAB_PAD = 1920
EMBV_PAD = 256


def _assemble_block(k, kb, wid_v, cid_v, bufw_v, stage_v, ctab_v, embv_v):
    def g_body(g, carry):
        off = g * 16
        lanes = off + lax.iota(jnp.int32, 16)
        wid = wid_v[k, pl.ds(off, 16)]
        cid = cid_v[k, pl.ds(off, 16)]
        for c in range(DIM):
            v = plsc.load_gather(
                bufw_v, [lanes, jnp.full((16,), c, jnp.int32)])
            stage_v[c, k, pl.ds(off, 16)] = v
        for c in range(TAGD + 1):
            v = plsc.load_gather(ctab_v, [cid * (TAGD + 1) + c])
            stage_v[DIM + c, k, pl.ds(off, 16)] = v
        msk = wid >= (VOCAB - 2)
        cnt = jnp.sum(jnp.where(msk, 1, 0).astype(jnp.int32))

        @pl.when(cnt > 0)
        def _fix():
            row = jnp.maximum(wid - (VOCAB - 3), 0) * DIM
            for c in range(DIM):
                v = plsc.load_gather(embv_v, [row + c], mask=msk)
                v = jnp.where(msk, v, 0.0)
                stage_v[c, k, pl.ds(off, 16)] = (
                    stage_v[c, k, pl.ds(off, 16)] + v)

        return carry

    lax.fori_loop(0, NG, g_body, 0)


def _sc_body(emb_fix_hbm, wid_hbm, cid_hbm, ctab_hbm, embv_hbm, out_hbm,
             wid_v, cid_v, bufw_v, stage_v, ctab_v, embv_v,
             ids_sem, gat_sem, out_sem):
    w = lax.axis_index("s") * NC + lax.axis_index("c")
    b0 = w * CB

    pltpu.sync_copy(ctab_hbm, ctab_v)
    pltpu.sync_copy(embv_hbm, embv_v)

    pltpu.sync_copy(wid_hbm.at[pl.ds(0, LG), pl.ds(b0, CB)], wid_v.at[0])
    pltpu.sync_copy(cid_hbm.at[pl.ds(0, LG), pl.ds(b0, CB)], cid_v.at[0])
    pltpu.async_copy(wid_hbm.at[pl.ds(LG, LG), pl.ds(b0, CB)], wid_v.at[1],
                     ids_sem.at[1])
    pltpu.async_copy(cid_hbm.at[pl.ds(LG, LG), pl.ds(b0, CB)], cid_v.at[1],
                     ids_sem.at[1])

    def drain_writes(lp):
        def c_body(c, carry):
            pltpu.make_async_copy(
                stage_v.at[c],
                out_hbm.at[pl.ds(c * SEQ + lp, LG), pl.ds(b0, CB)],
                out_sem).wait()
            return carry

        lax.fori_loop(0, OUTD, c_body, 0)

    def tile_body(t, carry):
        tb = t & 1
        nb = 1 - tb
        l0 = t * LG
        pltpu.async_copy(
            emb_fix_hbm.at[wid_v.at[tb, 0]], bufw_v.at[0], gat_sem.at[0])

        @pl.when(t > 0)
        def _drain():
            drain_writes((t - 1) * LG)

        def k_body(k, carry2):
            kb = k & 1

            @pl.when(k + 1 < LG)
            def _next_gather():
                pltpu.async_copy(
                    emb_fix_hbm.at[wid_v.at[tb, k + 1]],
                    bufw_v.at[1 - kb], gat_sem.at[1 - kb])

            pltpu.make_async_copy(
                emb_fix_hbm.at[wid_v.at[tb, k]], bufw_v.at[kb],
                gat_sem.at[kb]).wait()
            _assemble_block(k, kb, wid_v.at[tb], cid_v.at[tb],
                            bufw_v.at[kb], stage_v, ctab_v, embv_v)
            return carry2

        lax.fori_loop(0, LG, k_body, 0)

        @pl.when(t + 2 < NT)
        def _ids2():
            l2 = (t + 2) * LG
            pltpu.async_copy(wid_hbm.at[pl.ds(l2, LG), pl.ds(b0, CB)],
                             wid_v.at[tb], ids_sem.at[tb])
            pltpu.async_copy(cid_hbm.at[pl.ds(l2, LG), pl.ds(b0, CB)],
                             cid_v.at[tb], ids_sem.at[tb])

        def w_body(c, carry2):
            pltpu.async_copy(
                stage_v.at[c],
                out_hbm.at[pl.ds(c * SEQ + l0, LG), pl.ds(b0, CB)],
                out_sem)
            return carry2

        lax.fori_loop(0, OUTD, w_body, 0)

        @pl.when(t + 1 < NT)
        def _wait_ids():
            l1 = (t + 1) * LG
            pltpu.make_async_copy(
                wid_hbm.at[pl.ds(l1, LG), pl.ds(b0, CB)],
                wid_v.at[nb], ids_sem.at[nb]).wait()
            pltpu.make_async_copy(
                cid_hbm.at[pl.ds(l1, LG), pl.ds(b0, CB)],
                cid_v.at[nb], ids_sem.at[nb]).wait()

        return carry

    lax.fori_loop(0, NT, tile_body, 0)

    drain_writes((NT - 1) * LG)


@jax.jit
def _run(emb_fix, wid, cid, ctab, embv):
    mesh = plsc.VectorSubcoreMesh(core_axis_name="c", subcore_axis_name="s")
    f = pl.kernel(
        _sc_body,
        out_type=jax.ShapeDtypeStruct((OUTD * SEQ, BATCH), jnp.float32),
        mesh=mesh,
        compiler_params=pltpu.CompilerParams(needs_layout_passes=False),
        scratch_types=[
            pltpu.VMEM((2, LG, CB), jnp.int32),
            pltpu.VMEM((2, LG, CB), jnp.int32),
            pltpu.VMEM((2, CB, 2 * DIM), jnp.float32),
            pltpu.VMEM((OUTD, LG, CB), jnp.float32),
            pltpu.VMEM((CTAB_PAD,), jnp.float32),
            pltpu.VMEM((EMBV_PAD,), jnp.float32),
            pltpu.SemaphoreType.DMA((2,)),
            pltpu.SemaphoreType.DMA((2,)),
            pltpu.SemaphoreType.DMA,
        ],
    )
    return f(emb_fix, wid, cid, ctab, embv)


def kernel(word_ids, tag_ids, is_in, emb_fix, emb_v, tag_table):
    wid = word_ids.T.astype(jnp.int32)
    cid = (tag_ids + 100 * is_in).T.astype(jnp.int32)
    ctab = jnp.concatenate(
        [
            jnp.concatenate([tag_table, tag_table], axis=0),
            jnp.concatenate(
                [jnp.zeros((100, 1), jnp.float32),
                 jnp.ones((100, 1), jnp.float32)], axis=0),
        ],
        axis=1,
    ).reshape(-1)
    ctab = jnp.pad(ctab, (0, CTAB_PAD - ctab.shape[0]))
    embv = jnp.pad(emb_v.reshape(-1), (0, EMBV_PAD - 3 * DIM))
    emb_pad = jnp.pad(emb_fix, ((0, 0), (0, DIM)))
    out = _run(emb_pad, wid, cid, ctab, embv)
    return out.reshape(OUTD, SEQ, BATCH).transpose(1, 2, 0)


# speedup vs baseline: 1.0963x; 1.0963x over previous
import jax
import jax.numpy as jnp
from jax import lax
from jax.experimental import pallas as pl
from jax.experimental.pallas import tpu as pltpu
from jax.experimental.pallas import tpu_sc as plsc

VOCAB = 100000
DIM = 64
BATCH = 4096
SEQ = 200
TAGD = 8
OUTD = DIM + TAGD + 1

NC = 2
NS = 16
NW = NC * NS
CB = 128
LG = 8
NT = SEQ // LG
NG = CB // 16
CTAB_PAD = 1920
EMBV_PAD = 256


def _assemble_block(k, kb, wid_v, cid_v, bufw_v, stage_v, ctab_v, embv_v):
    def g_body(g, carry):
        off = g * 16
        lanes = off + lax.iota(jnp.int32, 16)
        wid = wid_v[k, pl.ds(off, 16)]
        cid = cid_v[k, pl.ds(off, 16)]
        for c in range(DIM):
            v = plsc.load_gather(
                bufw_v, [lanes, jnp.full((16,), c, jnp.int32)])
            stage_v[c, k, pl.ds(off, 16)] = v
        for c in range(TAGD + 1):
            v = plsc.load_gather(ctab_v, [cid * (TAGD + 1) + c])
            stage_v[DIM + c, k, pl.ds(off, 16)] = v
        msk = wid >= (VOCAB - 2)
        cnt = jnp.sum(jnp.where(msk, 1, 0).astype(jnp.int32))

        @pl.when(cnt > 0)
        def _fix():
            row = jnp.maximum(wid - (VOCAB - 3), 0) * DIM
            for c in range(DIM):
                v = plsc.load_gather(embv_v, [row + c], mask=msk)
                v = jnp.where(msk, v, 0.0)
                stage_v[c, k, pl.ds(off, 16)] = (
                    stage_v[c, k, pl.ds(off, 16)] + v)

        return carry

    lax.fori_loop(0, NG, g_body, 0)


def _sc_body(emb_fix_hbm, wid_hbm, cid_hbm, ctab_hbm, embv_hbm, out_hbm,
             wid_v, cid_v, bufw_v, stage_v, ctab_v, embv_v,
             ids_sem, gat_sem, out_sem):
    w = lax.axis_index("s") * NC + lax.axis_index("c")
    b0 = w * CB

    pltpu.sync_copy(ctab_hbm, ctab_v)
    pltpu.sync_copy(embv_hbm, embv_v)

    pltpu.sync_copy(wid_hbm.at[pl.ds(0, LG), pl.ds(b0, CB)], wid_v.at[0])
    pltpu.sync_copy(cid_hbm.at[pl.ds(0, LG), pl.ds(b0, CB)], cid_v.at[0])
    pltpu.async_copy(wid_hbm.at[pl.ds(LG, LG), pl.ds(b0, CB)], wid_v.at[1],
                     ids_sem.at[1])
    pltpu.async_copy(cid_hbm.at[pl.ds(LG, LG), pl.ds(b0, CB)], cid_v.at[1],
                     ids_sem.at[1])

    def drain_writes(lp):
        def c_body(c, carry):
            pltpu.make_async_copy(
                stage_v.at[c],
                out_hbm.at[pl.ds(c * SEQ + lp, LG), pl.ds(b0, CB)],
                out_sem).wait()
            return carry

        lax.fori_loop(0, OUTD, c_body, 0)

    def tile_body(t, carry):
        tb = t & 1
        nb = 1 - tb
        l0 = t * LG
        pltpu.async_copy(
            emb_fix_hbm.at[wid_v.at[tb, 0]], bufw_v.at[0], gat_sem.at[0])

        @pl.when(t > 0)
        def _drain():
            drain_writes((t - 1) * LG)

        for k in range(LG):
            kb = k % 2
            if k + 1 < LG:
                pltpu.async_copy(
                    emb_fix_hbm.at[wid_v.at[tb, k + 1]],
                    bufw_v.at[1 - kb], gat_sem.at[1 - kb])
            pltpu.make_async_copy(
                emb_fix_hbm.at[wid_v.at[tb, k]], bufw_v.at[kb],
                gat_sem.at[kb]).wait()
            _assemble_block(k, kb, wid_v.at[tb], cid_v.at[tb],
                            bufw_v.at[kb], stage_v, ctab_v, embv_v)

        @pl.when(t + 2 < NT)
        def _ids2():
            l2 = (t + 2) * LG
            pltpu.async_copy(wid_hbm.at[pl.ds(l2, LG), pl.ds(b0, CB)],
                             wid_v.at[tb], ids_sem.at[tb])
            pltpu.async_copy(cid_hbm.at[pl.ds(l2, LG), pl.ds(b0, CB)],
                             cid_v.at[tb], ids_sem.at[tb])

        def w_body(c, carry2):
            pltpu.async_copy(
                stage_v.at[c],
                out_hbm.at[pl.ds(c * SEQ + l0, LG), pl.ds(b0, CB)],
                out_sem)
            return carry2

        lax.fori_loop(0, OUTD, w_body, 0)

        @pl.when(t + 1 < NT)
        def _wait_ids():
            l1 = (t + 1) * LG
            pltpu.make_async_copy(
                wid_hbm.at[pl.ds(l1, LG), pl.ds(b0, CB)],
                wid_v.at[nb], ids_sem.at[nb]).wait()
            pltpu.make_async_copy(
                cid_hbm.at[pl.ds(l1, LG), pl.ds(b0, CB)],
                cid_v.at[nb], ids_sem.at[nb]).wait()

        return carry

    lax.fori_loop(0, NT, tile_body, 0)

    drain_writes((NT - 1) * LG)


@jax.jit
def _run(emb_fix, wid, cid, ctab, embv):
    mesh = plsc.VectorSubcoreMesh(core_axis_name="c", subcore_axis_name="s")
    f = pl.kernel(
        _sc_body,
        out_type=jax.ShapeDtypeStruct((OUTD * SEQ, BATCH), jnp.float32),
        mesh=mesh,
        compiler_params=pltpu.CompilerParams(needs_layout_passes=False),
        scratch_types=[
            pltpu.VMEM((2, LG, CB), jnp.int32),
            pltpu.VMEM((2, LG, CB), jnp.int32),
            pltpu.VMEM((2, CB, 2 * DIM), jnp.float32),
            pltpu.VMEM((OUTD, LG, CB), jnp.float32),
            pltpu.VMEM((CTAB_PAD,), jnp.float32),
            pltpu.VMEM((EMBV_PAD,), jnp.float32),
            pltpu.SemaphoreType.DMA((2,)),
            pltpu.SemaphoreType.DMA((2,)),
            pltpu.SemaphoreType.DMA,
        ],
    )
    return f(emb_fix, wid, cid, ctab, embv)


def kernel(word_ids, tag_ids, is_in, emb_fix, emb_v, tag_table):
    wid = word_ids.T.astype(jnp.int32)
    cid = (tag_ids + 100 * is_in).T.astype(jnp.int32)
    ctab = jnp.concatenate(
        [
            jnp.concatenate([tag_table, tag_table], axis=0),
            jnp.concatenate(
                [jnp.zeros((100, 1), jnp.float32),
                 jnp.ones((100, 1), jnp.float32)], axis=0),
        ],
        axis=1,
    ).reshape(-1)
    ctab = jnp.pad(ctab, (0, CTAB_PAD - ctab.shape[0]))
    embv = jnp.pad(emb_v.reshape(-1), (0, EMBV_PAD - 3 * DIM))
    emb_pad = jnp.pad(emb_fix, ((0, 0), (0, DIM)))
    out = _run(emb_pad, wid, cid, ctab, embv)
    return out.reshape(OUTD, SEQ, BATCH).transpose(1, 2, 0)


# speedup vs baseline: 1.3695x; 1.2492x over previous
import jax
import jax.numpy as jnp
from jax import lax
from jax.experimental import pallas as pl
from jax.experimental.pallas import tpu as pltpu
from jax.experimental.pallas import tpu_sc as plsc

VOCAB = 100000
DIM = 64
BATCH = 4096
SEQ = 200
TAGD = 8
OUTD = DIM + TAGD + 1

N = BATCH * SEQ
NC = 2
NS = 16
NW = NC * NS
PER_W = N // NW
C = 128
NCHUNK = PER_W // C
NG = C // 16
CTAB_PAD = 1920
EMBV_PAD = 256


def _assemble(off_n0, wid_v, cid_v, bufw_v, stage_v, ctab_v, embv_v):
    def g_body(g, carry):
        off = g * 16
        lanes = off + lax.iota(jnp.int32, 16)
        wid = wid_v[pl.ds(off, 16)]
        cid = cid_v[pl.ds(off, 16)]
        for e in range(16):
            r = off + e
            for k in range(DIM // 16):
                stage_v[r, pl.ds(k * 16, 16)] = bufw_v[r, pl.ds(k * 16, 16)]
        for c in range(TAGD + 1):
            vals = plsc.load_gather(ctab_v, [cid * (TAGD + 1) + c])
            plsc.store_scatter(
                stage_v, [lanes, jnp.full((16,), DIM + c, jnp.int32)], vals)
        msk = wid >= (VOCAB - 2)
        cnt = jnp.sum(jnp.where(msk, 1, 0).astype(jnp.int32))

        @pl.when(cnt > 0)
        def _fix():
            row = jnp.maximum(wid - (VOCAB - 3), 0) * DIM
            for c in range(DIM):
                v = plsc.load_gather(embv_v, [row + c], mask=msk)
                plsc.addupdate_scatter(
                    stage_v, [lanes, jnp.full((16,), c, jnp.int32)],
                    v, mask=msk)

        return carry

    lax.fori_loop(0, NG, g_body, 0)


def _sc_body(emb_fix_hbm, wid_hbm, cid_hbm, ctab_hbm, embv_hbm, out_hbm,
             wid_v, cid_v, bufw_v, stage_v, ctab_v, embv_v,
             ids_sem, gat_sem, out_sem):
    w = lax.axis_index("s") * NC + lax.axis_index("c")
    base = w * PER_W

    pltpu.sync_copy(ctab_hbm, ctab_v)
    pltpu.sync_copy(embv_hbm, embv_v)

    pltpu.sync_copy(wid_hbm.at[pl.ds(base, C)], wid_v.at[0])
    pltpu.sync_copy(cid_hbm.at[pl.ds(base, C)], cid_v.at[0])
    pltpu.async_copy(emb_fix_hbm.at[wid_v.at[0]], bufw_v.at[0],
                     gat_sem.at[0])
    pltpu.async_copy(wid_hbm.at[pl.ds(base + C, C)], wid_v.at[1],
                     ids_sem.at[1])
    pltpu.async_copy(cid_hbm.at[pl.ds(base + C, C)], cid_v.at[1],
                     ids_sem.at[1])

    def pair_body(p, carry):
        for b in (0, 1):
            j = 2 * p + b
            nb = 1 - b
            n0 = base + j * C
            pltpu.make_async_copy(
                emb_fix_hbm.at[wid_v.at[b]], bufw_v.at[b],
                gat_sem.at[b]).wait()

            @pl.when(j + 1 < NCHUNK)
            def _next_gather():
                pltpu.make_async_copy(
                    wid_hbm.at[pl.ds(n0 + C, C)], wid_v.at[nb],
                    ids_sem.at[nb]).wait()
                pltpu.make_async_copy(
                    cid_hbm.at[pl.ds(n0 + C, C)], cid_v.at[nb],
                    ids_sem.at[nb]).wait()
                pltpu.async_copy(
                    emb_fix_hbm.at[wid_v.at[nb]], bufw_v.at[nb],
                    gat_sem.at[nb])

            @pl.when(j >= 2)
            def _wait_write():
                pltpu.make_async_copy(
                    stage_v.at[b], out_hbm.at[pl.ds(n0 - 2 * C, C)],
                    out_sem.at[b]).wait()

            _assemble(n0, wid_v.at[b], cid_v.at[b], bufw_v.at[b],
                      stage_v.at[b], ctab_v, embv_v)

            @pl.when(j + 2 < NCHUNK)
            def _next_ids():
                pltpu.async_copy(
                    wid_hbm.at[pl.ds(n0 + 2 * C, C)], wid_v.at[b],
                    ids_sem.at[b])
                pltpu.async_copy(
                    cid_hbm.at[pl.ds(n0 + 2 * C, C)], cid_v.at[b],
                    ids_sem.at[b])

            pltpu.async_copy(stage_v.at[b], out_hbm.at[pl.ds(n0, C)],
                             out_sem.at[b])
        return carry

    lax.fori_loop(0, NCHUNK // 2, pair_body, 0)

    for b in (0, 1):
        n_last = base + (NCHUNK - 2 + b) * C
        pltpu.make_async_copy(
            stage_v.at[b], out_hbm.at[pl.ds(n_last, C)],
            out_sem.at[b]).wait()


@jax.jit
def _run(emb_fix, wid, cid, ctab, embv):
    mesh = plsc.VectorSubcoreMesh(core_axis_name="c", subcore_axis_name="s")
    f = pl.kernel(
        _sc_body,
        out_type=jax.ShapeDtypeStruct((N, OUTD), jnp.float32),
        mesh=mesh,
        compiler_params=pltpu.CompilerParams(needs_layout_passes=False),
        scratch_types=[
            pltpu.VMEM((2, C), jnp.int32),
            pltpu.VMEM((2, C), jnp.int32),
            pltpu.VMEM((2, C, 2 * DIM), jnp.float32),
            pltpu.VMEM((2, C, OUTD), jnp.float32),
            pltpu.VMEM((CTAB_PAD,), jnp.float32),
            pltpu.VMEM((EMBV_PAD,), jnp.float32),
            pltpu.SemaphoreType.DMA((2,)),
            pltpu.SemaphoreType.DMA((2,)),
            pltpu.SemaphoreType.DMA((2,)),
        ],
    )
    return f(emb_fix, wid, cid, ctab, embv)


TBLK = 4096


def _to_feature_major(kout):
    def body(i_ref, o_ref):
        o_ref[...] = i_ref[...].T

    return pl.pallas_call(
        body,
        grid=(N // TBLK,),
        in_specs=[pl.BlockSpec((TBLK, OUTD), lambda i: (i, 0))],
        out_specs=pl.BlockSpec((OUTD, TBLK), lambda i: (0, i)),
        out_shape=jax.ShapeDtypeStruct((OUTD, N), jnp.float32),
    )(kout)


def kernel(word_ids, tag_ids, is_in, emb_fix, emb_v, tag_table):
    wid = word_ids.T.reshape(-1).astype(jnp.int32)
    cid = (tag_ids + 100 * is_in).T.reshape(-1).astype(jnp.int32)
    ctab = jnp.concatenate(
        [
            jnp.concatenate([tag_table, tag_table], axis=0),
            jnp.concatenate(
                [jnp.zeros((100, 1), jnp.float32),
                 jnp.ones((100, 1), jnp.float32)], axis=0),
        ],
        axis=1,
    ).reshape(-1)
    ctab = jnp.pad(ctab, (0, CTAB_PAD - ctab.shape[0]))
    embv = jnp.pad(emb_v.reshape(-1), (0, EMBV_PAD - 3 * DIM))
    emb_pad = jnp.pad(emb_fix, ((0, 0), (0, DIM)))
    out = _run(emb_pad, wid, cid, ctab, embv)
    outt = _to_feature_major(out)
    return outt.reshape(OUTD, SEQ, BATCH).transpose(1, 2, 0)


# speedup vs baseline: 2.0070x; 1.4654x over previous
import jax
import jax.numpy as jnp
from jax import lax
from jax.experimental import pallas as pl
from jax.experimental.pallas import tpu as pltpu
from jax.experimental.pallas import tpu_sc as plsc

VOCAB = 100000
DIM = 64
BATCH = 4096
SEQ = 200
TAGD = 8
OUTD = DIM + TAGD + 1

N = BATCH * SEQ
NC = 2
NS = 16
NW = NC * NS
PER_W = N // NW
C = 128
NCHUNK = PER_W // C
NG = C // 16
CTAB_PAD = 1920
EMBV_PAD = 256


def _assemble(off_n0, wid_v, cid_v, bufw_v, stage_v, ctab_v, embv_v):
    def g_body(g, carry):
        off = g * 16
        lanes = off + lax.iota(jnp.int32, 16)
        wid = wid_v[pl.ds(off, 16)]
        cid = cid_v[pl.ds(off, 16)]
        for e in range(16):
            r = off + e
            for k in range(DIM // 16):
                stage_v[r, pl.ds(k * 16, 16)] = bufw_v[r, pl.ds(k * 16, 16)]
        for c in range(TAGD + 1):
            vals = plsc.load_gather(ctab_v, [cid * (TAGD + 1) + c])
            plsc.store_scatter(
                stage_v, [lanes, jnp.full((16,), DIM + c, jnp.int32)], vals)
        msk = wid >= (VOCAB - 2)
        cnt = jnp.sum(jnp.where(msk, 1, 0).astype(jnp.int32))

        @pl.when(cnt > 0)
        def _fix():
            row = jnp.maximum(wid - (VOCAB - 3), 0) * DIM
            for c in range(DIM):
                v = plsc.load_gather(embv_v, [row + c], mask=msk)
                plsc.addupdate_scatter(
                    stage_v, [lanes, jnp.full((16,), c, jnp.int32)],
                    v, mask=msk)

        return carry

    lax.fori_loop(0, NG, g_body, 0)


def _sc_body(emb_fix_hbm, wid_hbm, cid_hbm, ctab_hbm, embv_hbm, out_hbm,
             wid_v, cid_v, bufw_v, stage_v, ctab_v, embv_v,
             ids_sem, gat_sem, out_sem):
    w = lax.axis_index("s") * NC + lax.axis_index("c")
    base = w * PER_W

    pltpu.sync_copy(ctab_hbm, ctab_v)
    pltpu.sync_copy(embv_hbm, embv_v)

    pltpu.sync_copy(wid_hbm.at[pl.ds(base, C)], wid_v.at[0])
    pltpu.sync_copy(cid_hbm.at[pl.ds(base, C)], cid_v.at[0])
    pltpu.async_copy(emb_fix_hbm.at[wid_v.at[0]], bufw_v.at[0],
                     gat_sem.at[0])
    pltpu.async_copy(wid_hbm.at[pl.ds(base + C, C)], wid_v.at[1],
                     ids_sem.at[1])
    pltpu.async_copy(cid_hbm.at[pl.ds(base + C, C)], cid_v.at[1],
                     ids_sem.at[1])

    def pair_body(p, carry):
        for b in (0, 1):
            j = 2 * p + b
            nb = 1 - b
            n0 = base + j * C
            pltpu.make_async_copy(
                emb_fix_hbm.at[wid_v.at[b]], bufw_v.at[b],
                gat_sem.at[b]).wait()

            @pl.when(j + 1 < NCHUNK)
            def _next_gather():
                pltpu.make_async_copy(
                    wid_hbm.at[pl.ds(n0 + C, C)], wid_v.at[nb],
                    ids_sem.at[nb]).wait()
                pltpu.make_async_copy(
                    cid_hbm.at[pl.ds(n0 + C, C)], cid_v.at[nb],
                    ids_sem.at[nb]).wait()
                pltpu.async_copy(
                    emb_fix_hbm.at[wid_v.at[nb]], bufw_v.at[nb],
                    gat_sem.at[nb])

            @pl.when(j >= 2)
            def _wait_write():
                pltpu.make_async_copy(
                    stage_v.at[b], out_hbm.at[pl.ds(n0 - 2 * C, C)],
                    out_sem.at[b]).wait()

            _assemble(n0, wid_v.at[b], cid_v.at[b], bufw_v.at[b],
                      stage_v.at[b], ctab_v, embv_v)

            @pl.when(j + 2 < NCHUNK)
            def _next_ids():
                pltpu.async_copy(
                    wid_hbm.at[pl.ds(n0 + 2 * C, C)], wid_v.at[b],
                    ids_sem.at[b])
                pltpu.async_copy(
                    cid_hbm.at[pl.ds(n0 + 2 * C, C)], cid_v.at[b],
                    ids_sem.at[b])

            pltpu.async_copy(stage_v.at[b], out_hbm.at[pl.ds(n0, C)],
                             out_sem.at[b])
        return carry

    lax.fori_loop(0, NCHUNK // 2, pair_body, 0)

    for b in (0, 1):
        n_last = base + (NCHUNK - 2 + b) * C
        pltpu.make_async_copy(
            stage_v.at[b], out_hbm.at[pl.ds(n_last, C)],
            out_sem.at[b]).wait()


@jax.jit
def _run(emb_fix, wid, cid, ctab, embv):
    mesh = plsc.VectorSubcoreMesh(core_axis_name="c", subcore_axis_name="s")
    f = pl.kernel(
        _sc_body,
        out_type=jax.ShapeDtypeStruct((N, OUTD), jnp.float32),
        mesh=mesh,
        compiler_params=pltpu.CompilerParams(needs_layout_passes=False),
        scratch_types=[
            pltpu.VMEM((2, C), jnp.int32),
            pltpu.VMEM((2, C), jnp.int32),
            pltpu.VMEM((2, C, 2 * DIM), jnp.float32),
            pltpu.VMEM((2, C, OUTD), jnp.float32),
            pltpu.VMEM((CTAB_PAD,), jnp.float32),
            pltpu.VMEM((EMBV_PAD,), jnp.float32),
            pltpu.SemaphoreType.DMA((2,)),
            pltpu.SemaphoreType.DMA((2,)),
            pltpu.SemaphoreType.DMA((2,)),
        ],
    )
    return f(emb_fix, wid, cid, ctab, embv)


def kernel(word_ids, tag_ids, is_in, emb_fix, emb_v, tag_table):
    wid = word_ids.T.reshape(-1).astype(jnp.int32)
    cid = (tag_ids + 100 * is_in).T.reshape(-1).astype(jnp.int32)
    ctab = jnp.concatenate(
        [
            jnp.concatenate([tag_table, tag_table], axis=0),
            jnp.concatenate(
                [jnp.zeros((100, 1), jnp.float32),
                 jnp.ones((100, 1), jnp.float32)], axis=0),
        ],
        axis=1,
    ).reshape(-1)
    ctab = jnp.pad(ctab, (0, CTAB_PAD - ctab.shape[0]))
    embv = jnp.pad(emb_v.reshape(-1), (0, EMBV_PAD - 3 * DIM))
    emb_pad = jnp.pad(emb_fix, ((0, 0), (0, DIM)))
    out = _run(emb_pad, wid, cid, ctab, embv)
    return out.reshape(SEQ, BATCH, OUTD)
